# baseline (speedup 1.0000x reference)
import functools

import jax
import jax.numpy as jnp
import numpy as np
from jax import lax
from jax.experimental import pallas as pl
from jax.experimental.pallas import tpu as pltpu
from jax.experimental.pallas import tpu_sc as plsc

N_ATOMS = 50000
N_EDGES = 800000
CUTOFF = 5.0
WIDTH = 0.5

NC = 2
NS = 16
NW = NC * NS

E_PAD = 819200
EW = E_PAD // NW
CHUNK = 1024
IDXW = 128
IDXROWS = CHUNK // IDXW
NCHUNK = EW // CHUNK

VCOL = 512
VROWS = E_PAD // VCOL
CROWS = CHUNK // VCOL

N_PAD = 50176
ROWS_PER_TILE = N_PAD // NS

BROW = 8


def _edge_feat_body(v_ref, p_ref):
    vx = v_ref[0]
    vy = v_ref[1]
    vz = v_ref[2]
    r2 = vx * vx + vy * vy + vz * vz + 1e-12
    inv_r2 = 1.0 / r2
    r = jnp.sqrt(r2)
    theta = (jnp.pi / CUTOFF) * jnp.minimum(r, CUTOFF)
    u = theta - (jnp.pi / 2)
    u2 = u * u
    sin_u = u * (1.0 + u2 * (-1.6666667e-1 + u2 * (8.3333331e-3
            + u2 * (-1.9840874e-4 + u2 * 2.7525562e-6))))
    cos_u = 1.0 + u2 * (-0.5 + u2 * (4.1666668e-2 + u2 * (-1.3888889e-3
            + u2 * (2.4801587e-5 + u2 * -2.7557319e-7))))
    s1 = cos_u
    c1 = -sin_u
    two_c1 = 2.0 * c1
    sines = [s1, two_c1 * s1]
    for _ in range(6):
        sines.append(two_c1 * sines[-1] - sines[-2])
    s5 = sines[4]
    fc = jnp.where(r < CUTOFF - WIDTH, 1.0, s5 * s5)
    fc = jnp.where(r < CUTOFF, fc, 0.0)
    scale = inv_r2 * fc
    rows = jnp.stack(
        [s * scale for s in sines]
        + [vx, vy, vz, jnp.zeros((BROW, VCOL), jnp.float32)], axis=1)
    rows = jnp.concatenate(
        [rows, jnp.zeros((BROW, 4, VCOL), jnp.float32)], axis=1)
    p_ref[...] = rows.reshape(BROW * 16, VCOL)


def _edge_feats(vt3):
    return pl.pallas_call(
        _edge_feat_body,
        grid=(VROWS // BROW,),
        in_specs=[pl.BlockSpec((3, BROW, VCOL), lambda i: (0, i, 0))],
        out_specs=pl.BlockSpec((BROW * 16, VCOL), lambda i: (i, 0)),
        out_shape=jax.ShapeDtypeStruct((VROWS * 16, VCOL), jnp.float32),
    )(vt3)


def _idx_pad_body(c_ref, n_ref, co_ref, no_ref):
    co_ref[...] = c_ref[...]
    no_ref[...] = n_ref[...]


def _idx_pad(cen, nei):
    rin = N_EDGES // (5 * IDXW)
    rout = E_PAD // (5 * IDXW)
    br = 10
    nblk = rin // br
    c3 = cen.reshape(rin, 5, IDXW)
    n3 = nei.reshape(rin, 5, IDXW)
    outs = pl.pallas_call(
        _idx_pad_body,
        grid=(rout // br,),
        in_specs=[
            pl.BlockSpec((br, 5, IDXW),
                         lambda i: (jnp.minimum(i, nblk - 1), 0, 0)),
            pl.BlockSpec((br, 5, IDXW),
                         lambda i: (jnp.minimum(i, nblk - 1), 0, 0)),
        ],
        out_specs=[pl.BlockSpec((br, 5, IDXW), lambda i: (i, 0, 0)),
                   pl.BlockSpec((br, 5, IDXW), lambda i: (i, 0, 0))],
        out_shape=[jax.ShapeDtypeStruct((rout, 5, IDXW), jnp.int32),
                   jax.ShapeDtypeStruct((rout, 5, IDXW), jnp.int32)],
    )(c3, n3)
    return (outs[0].reshape(E_PAD // IDXW, IDXW),
            outs[1].reshape(E_PAD // IDXW, IDXW))


def _sc_body(p_hbm, cen_hbm, nei_hbm, sp_hbm, g_hbm, out_hbm,
             sp_v, g_v, p_v0, p_v1, cen_v0, cen_v1, nei_v0, nei_v1,
             stg_v, acc_sh, dsem0, dsem1, ssem):
    cid = lax.axis_index("c")
    sid = lax.axis_index("s")
    wid = sid * NC + cid

    pltpu.sync_copy(sp_hbm, sp_v)
    pltpu.sync_copy(g_hbm, g_v)

    zero16 = jnp.zeros((16,), jnp.float32)

    def _zero_body(i, _):
        stg_v[i, :] = zero16
        return 0

    lax.fori_loop(0, CHUNK, _zero_body, 0)

    quarter = ROWS_PER_TILE // 4
    r0 = sid * ROWS_PER_TILE
    for k in range(4):
        pltpu.sync_copy(stg_v.at[pl.ds(0, quarter)],
                        acc_sh.at[pl.ds(r0 + k * quarter, quarter)])
    plsc.subcore_barrier()

    lane = lax.iota(jnp.int32, 16)
    bufs = ((p_v0, cen_v0, nei_v0, dsem0), (p_v1, cen_v1, nei_v1, dsem1))

    def _fire(it, buf):
        p_v, cen_v, nei_v, sem = buf
        base = pl.multiple_of(wid * EW + it * CHUNK, CHUNK)
        pr0 = pl.multiple_of((base // VCOL) * 16, CROWS * 16)
        row0 = pl.multiple_of(base // IDXW, IDXROWS)
        pltpu.make_async_copy(
            p_hbm.at[pl.ds(pr0, CROWS * 16)], p_v, sem).start()
        pltpu.make_async_copy(
            cen_hbm.at[pl.ds(row0, IDXROWS)], cen_v, sem).start()
        pltpu.make_async_copy(
            nei_hbm.at[pl.ds(row0, IDXROWS)], nei_v, sem).start()

    def _wait(buf):
        p_v, cen_v, nei_v, sem = buf
        pltpu.make_async_copy(
            p_hbm.at[pl.ds(0, CROWS * 16)], p_v, sem).wait()
        pltpu.make_async_copy(
            cen_hbm.at[pl.ds(0, IDXROWS)], cen_v, sem).wait()
        pltpu.make_async_copy(
            nei_hbm.at[pl.ds(0, IDXROWS)], nei_v, sem).wait()

    def _process(buf):
        p_v, cen_v, nei_v, _ = buf

        def _a_body(a, _):
            def _r_body(r8, _):
                def _q_body(q, _):
                    colv = r8 * IDXW + q * 16
                    j = a * 4 + r8
                    off = q * 16
                    e0 = a * VCOL + colv
                    nvec = nei_v[j, pl.ds(off, 16)]
                    cvec = cen_v[j, pl.ds(off, 16)]
                    t_w = plsc.load_gather(
                        sp_v, [lax.shift_right_logical(nvec, 2)])
                    s_w = plsc.load_gather(
                        sp_v, [lax.shift_right_logical(cvec, 2)])
                    t_sp = lax.shift_right_logical(
                        t_w, lax.shift_left(nvec & 3, 3)) & 7
                    s_sp = lax.shift_right_logical(
                        s_w, lax.shift_left(cvec & 3, 3)) & 7
                    u24 = (t_sp * 5 + s_sp) * 24
                    col = pl.ds(colv, 16)
                    a16 = a * 16
                    rfc = [p_v[a16 + i, col] for i in range(8)]
                    ys = []
                    for b in range(3):
                        acc = rfc[0] * plsc.load_gather(g_v, [u24 + (b * 8)])
                        for i in range(1, 8):
                            acc = acc + rfc[i] * plsc.load_gather(
                                g_v, [u24 + (b * 8 + i)])
                        ys.append(acc)
                    rows = e0 + lane
                    for m in range(3):
                        d = p_v[a16 + 8 + m, col]
                        for b in range(3):
                            plsc.store_scatter(
                                stg_v,
                                [rows, jnp.full((16,), m * 3 + b, jnp.int32)],
                                d * ys[b])
                    return 0

                lax.fori_loop(0, IDXW // 16, _q_body, 0)
                return 0

            lax.fori_loop(0, VCOL // IDXW, _r_body, 0)
            return 0

        lax.fori_loop(0, CROWS, _a_body, 0)

        descs = []
        for j in range(IDXROWS):
            d = pltpu.make_async_copy(stg_v.at[pl.ds(j * IDXW, IDXW)],
                                      acc_sh.at[cen_v.at[j]], ssem)
            d.start(add=True)
            descs.append(d)
        for d in descs:
            d.wait()

    _fire(0, bufs[0])

    def _chunk_body(it, _):
        @pl.when(lax.rem(it, 2) == 0)
        def _():
            _wait(bufs[0])

            @pl.when(it + 1 < NCHUNK)
            def _():
                _fire(it + 1, bufs[1])

            _process(bufs[0])

        @pl.when(lax.rem(it, 2) == 1)
        def _():
            _wait(bufs[1])

            @pl.when(it + 1 < NCHUNK)
            def _():
                _fire(it + 1, bufs[0])

            _process(bufs[1])

        return 0

    lax.fori_loop(0, NCHUNK, _chunk_body, 0)

    plsc.subcore_barrier()
    pltpu.sync_copy(acc_sh.at[pl.ds(r0, ROWS_PER_TILE)],
                    out_hbm.at[cid, pl.ds(r0, ROWS_PER_TILE)])


@functools.cache
def _make_sc_scatter():
    return pl.kernel(
        _sc_body,
        out_type=jax.ShapeDtypeStruct((NC, N_PAD, 16), jnp.float32),
        mesh=plsc.VectorSubcoreMesh(core_axis_name="c", subcore_axis_name="s",
                                    num_cores=NC, num_subcores=NS),
        scratch_types=[
            pltpu.VMEM((N_PAD // 4,), jnp.int32),
            pltpu.VMEM((640,), jnp.float32),
            pltpu.VMEM((CROWS * 16, VCOL), jnp.float32),
            pltpu.VMEM((CROWS * 16, VCOL), jnp.float32),
            pltpu.VMEM((IDXROWS, IDXW), jnp.int32),
            pltpu.VMEM((IDXROWS, IDXW), jnp.int32),
            pltpu.VMEM((IDXROWS, IDXW), jnp.int32),
            pltpu.VMEM((IDXROWS, IDXW), jnp.int32),
            pltpu.VMEM((CHUNK, 16), jnp.float32),
            pltpu.VMEM_SHARED((N_PAD, 16), jnp.float32),
            pltpu.SemaphoreType.DMA,
            pltpu.SemaphoreType.DMA,
            pltpu.SemaphoreType.DMA,
        ],
        compiler_params=pltpu.CompilerParams(needs_layout_passes=False,
                                             use_tc_tiling_on_sc=False),
    )


def _combine_body(p_ref, o_ref):
    s = p_ref[0] + p_ref[1]
    o_ref[...] = s[:, :9]


def _combine(parts):
    bn = 2000
    return pl.pallas_call(
        _combine_body,
        grid=(N_ATOMS // bn,),
        in_specs=[pl.BlockSpec((2, bn, 16), lambda i: (0, i, 0))],
        out_specs=pl.BlockSpec((bn, 9), lambda i: (i, 0)),
        out_shape=jax.ShapeDtypeStruct((N_ATOMS, 9), jnp.float32),
    )(parts)


def kernel(interatomic_vectors, centers, neighbors, species, sample_values,
           neighbor_embed, center_embed, W):
    del sample_values
    ce4 = center_embed.reshape(5, 8, 4)
    w4 = W.reshape(3, 8, 4)
    g = jnp.einsum("tj,sij,bij->tsbi", neighbor_embed, ce4, w4)
    g_flat = jnp.pad(g.reshape(-1), (0, 640 - 600)).astype(jnp.float32)

    vt3 = jnp.pad(interatomic_vectors.T,
                  ((0, 0), (0, E_PAD - N_EDGES))).reshape(3, VROWS, VCOL)
    cen2, nei2 = _idx_pad(centers, neighbors)
    sp4 = jnp.pad(species, (0, N_PAD - N_ATOMS)).reshape(N_PAD // 4, 4)
    sp_packed = (sp4[:, 0] | (sp4[:, 1] << 8) | (sp4[:, 2] << 16)
                 | (sp4[:, 3] << 24))

    p = _edge_feats(vt3)
    parts = _make_sc_scatter()(p, cen2, nei2, sp_packed, g_flat)
    return _combine(parts).reshape(N_ATOMS, 3, 3)


# speedup vs baseline: 1.1578x; 1.1578x over previous
import functools

import jax
import jax.numpy as jnp
import numpy as np
from jax import lax
from jax.experimental import pallas as pl
from jax.experimental.pallas import tpu as pltpu
from jax.experimental.pallas import tpu_sc as plsc

N_ATOMS = 50000
N_EDGES = 800000
CUTOFF = 5.0
WIDTH = 0.5

NC = 2
NS = 16
NW = NC * NS

E_PAD = 819200
EW = E_PAD // NW
CHUNK = 1024
IDXW = 128
IDXROWS = CHUNK // IDXW
NCHUNK = EW // CHUNK

VCOL = 512
VROWS = E_PAD // VCOL
CROWS = CHUNK // VCOL

N_PAD = 50176
ROWS_PER_TILE = N_PAD // NS

BROW = 8


def _edge_feat_body(v_ref, p_ref):
    vx = v_ref[0]
    vy = v_ref[1]
    vz = v_ref[2]
    r2 = vx * vx + vy * vy + vz * vz + 1e-12
    inv_r2 = 1.0 / r2
    r = jnp.sqrt(r2)
    theta = (jnp.pi / CUTOFF) * jnp.minimum(r, CUTOFF)
    u = theta - (jnp.pi / 2)
    u2 = u * u
    sin_u = u * (1.0 + u2 * (-1.6666667e-1 + u2 * (8.3333331e-3
            + u2 * (-1.9840874e-4 + u2 * 2.7525562e-6))))
    cos_u = 1.0 + u2 * (-0.5 + u2 * (4.1666668e-2 + u2 * (-1.3888889e-3
            + u2 * (2.4801587e-5 + u2 * -2.7557319e-7))))
    s1 = cos_u
    c1 = -sin_u
    two_c1 = 2.0 * c1
    sines = [s1, two_c1 * s1]
    for _ in range(6):
        sines.append(two_c1 * sines[-1] - sines[-2])
    s5 = sines[4]
    fc = jnp.where(r < CUTOFF - WIDTH, 1.0, s5 * s5)
    fc = jnp.where(r < CUTOFF, fc, 0.0)
    scale = inv_r2 * fc
    rows = jnp.stack(
        [s * scale for s in sines]
        + [vx, vy, vz, jnp.zeros((BROW, VCOL), jnp.float32)], axis=1)
    rows = jnp.concatenate(
        [rows, jnp.zeros((BROW, 4, VCOL), jnp.float32)], axis=1)
    p_ref[...] = rows.reshape(BROW * 16, VCOL)


def _edge_feats(vt3):
    return pl.pallas_call(
        _edge_feat_body,
        grid=(VROWS // BROW,),
        in_specs=[pl.BlockSpec((3, BROW, VCOL), lambda i: (0, i, 0))],
        out_specs=pl.BlockSpec((BROW * 16, VCOL), lambda i: (i, 0)),
        out_shape=jax.ShapeDtypeStruct((VROWS * 16, VCOL), jnp.float32),
    )(vt3)


def _sc_body(p_hbm, cen_hbm, nei_hbm, sp_hbm, g_hbm, out_hbm,
             sp_v, g_v, p_v0, p_v1, cen_v0, cen_v1, nei_v0, nei_v1,
             stg_v, acc_sh, dsem0, dsem1, ssem):
    cid = lax.axis_index("c")
    sid = lax.axis_index("s")
    wid = sid * NC + cid

    pltpu.sync_copy(sp_hbm, sp_v)
    pltpu.sync_copy(g_hbm, g_v)

    zero16 = jnp.zeros((16,), jnp.float32)

    def _zero_body(i, _):
        stg_v[i, :] = zero16
        return 0

    lax.fori_loop(0, CHUNK, _zero_body, 0)

    quarter = ROWS_PER_TILE // 4
    r0 = sid * ROWS_PER_TILE
    for k in range(4):
        pltpu.sync_copy(stg_v.at[pl.ds(0, quarter)],
                        acc_sh.at[pl.ds(r0 + k * quarter, quarter)])
    plsc.subcore_barrier()

    lane = lax.iota(jnp.int32, 16)
    bufs = ((p_v0, cen_v0, nei_v0, dsem0), (p_v1, cen_v1, nei_v1, dsem1))

    def _fire(it, buf):
        p_v, cen_v, nei_v, sem = buf
        base = pl.multiple_of(wid * EW + it * CHUNK, CHUNK)
        pr0 = pl.multiple_of((base // VCOL) * 16, CROWS * 16)
        row0 = pl.multiple_of(base // IDXW, IDXROWS)
        pltpu.make_async_copy(
            p_hbm.at[pl.ds(pr0, CROWS * 16)], p_v, sem).start()
        pltpu.make_async_copy(
            cen_hbm.at[pl.ds(row0, IDXROWS)], cen_v, sem).start()
        pltpu.make_async_copy(
            nei_hbm.at[pl.ds(row0, IDXROWS)], nei_v, sem).start()

    def _wait(buf):
        p_v, cen_v, nei_v, sem = buf
        pltpu.make_async_copy(
            p_hbm.at[pl.ds(0, CROWS * 16)], p_v, sem).wait()
        pltpu.make_async_copy(
            cen_hbm.at[pl.ds(0, IDXROWS)], cen_v, sem).wait()
        pltpu.make_async_copy(
            nei_hbm.at[pl.ds(0, IDXROWS)], nei_v, sem).wait()

    def _process(buf):
        p_v, cen_v, nei_v, _ = buf

        def _a_body(a, _):
            def _r_body(r8, _):
                def _q_body(q, _):
                    colv = r8 * IDXW + q * 16
                    j = a * 4 + r8
                    off = q * 16
                    e0 = a * VCOL + colv
                    nvec = nei_v[j, pl.ds(off, 16)]
                    cvec = cen_v[j, pl.ds(off, 16)]
                    t_w = plsc.load_gather(
                        sp_v, [lax.shift_right_logical(nvec, 2)])
                    s_w = plsc.load_gather(
                        sp_v, [lax.shift_right_logical(cvec, 2)])
                    t_sp = lax.shift_right_logical(
                        t_w, lax.shift_left(nvec & 3, 3)) & 7
                    s_sp = lax.shift_right_logical(
                        s_w, lax.shift_left(cvec & 3, 3)) & 7
                    u24 = (t_sp * 5 + s_sp) * 24
                    col = pl.ds(colv, 16)
                    a16 = a * 16
                    rfc = [p_v[a16 + i, col] for i in range(8)]
                    ys = []
                    for b in range(3):
                        acc = rfc[0] * plsc.load_gather(g_v, [u24 + (b * 8)])
                        for i in range(1, 8):
                            acc = acc + rfc[i] * plsc.load_gather(
                                g_v, [u24 + (b * 8 + i)])
                        ys.append(acc)
                    rows = e0 + lane
                    for m in range(3):
                        d = p_v[a16 + 8 + m, col]
                        for b in range(3):
                            plsc.store_scatter(
                                stg_v,
                                [rows, jnp.full((16,), m * 3 + b, jnp.int32)],
                                d * ys[b])
                    return 0

                lax.fori_loop(0, IDXW // 16, _q_body, 0)
                return 0

            lax.fori_loop(0, VCOL // IDXW, _r_body, 0)
            return 0

        lax.fori_loop(0, CROWS, _a_body, 0)

        descs = []
        for j in range(IDXROWS):
            d = pltpu.make_async_copy(stg_v.at[pl.ds(j * IDXW, IDXW)],
                                      acc_sh.at[cen_v.at[j]], ssem)
            d.start(add=True)
            descs.append(d)
        for d in descs:
            d.wait()

    _fire(0, bufs[0])

    def _chunk_body(it, _):
        @pl.when(lax.rem(it, 2) == 0)
        def _():
            _wait(bufs[0])

            @pl.when(it + 1 < NCHUNK)
            def _():
                _fire(it + 1, bufs[1])

            _process(bufs[0])

        @pl.when(lax.rem(it, 2) == 1)
        def _():
            _wait(bufs[1])

            @pl.when(it + 1 < NCHUNK)
            def _():
                _fire(it + 1, bufs[0])

            _process(bufs[1])

        return 0

    lax.fori_loop(0, NCHUNK, _chunk_body, 0)

    plsc.subcore_barrier()
    pltpu.sync_copy(acc_sh.at[pl.ds(r0, ROWS_PER_TILE)],
                    out_hbm.at[cid, pl.ds(r0, ROWS_PER_TILE)])


@functools.cache
def _make_sc_scatter():
    return pl.kernel(
        _sc_body,
        out_type=jax.ShapeDtypeStruct((NC, N_PAD, 16), jnp.float32),
        mesh=plsc.VectorSubcoreMesh(core_axis_name="c", subcore_axis_name="s",
                                    num_cores=NC, num_subcores=NS),
        scratch_types=[
            pltpu.VMEM((N_PAD // 4,), jnp.int32),
            pltpu.VMEM((640,), jnp.float32),
            pltpu.VMEM((CROWS * 16, VCOL), jnp.float32),
            pltpu.VMEM((CROWS * 16, VCOL), jnp.float32),
            pltpu.VMEM((IDXROWS, IDXW), jnp.int32),
            pltpu.VMEM((IDXROWS, IDXW), jnp.int32),
            pltpu.VMEM((IDXROWS, IDXW), jnp.int32),
            pltpu.VMEM((IDXROWS, IDXW), jnp.int32),
            pltpu.VMEM((CHUNK, 16), jnp.float32),
            pltpu.VMEM_SHARED((N_PAD, 16), jnp.float32),
            pltpu.SemaphoreType.DMA,
            pltpu.SemaphoreType.DMA,
            pltpu.SemaphoreType.DMA,
        ],
        compiler_params=pltpu.CompilerParams(needs_layout_passes=False,
                                             use_tc_tiling_on_sc=False),
    )


def _combine_body(p_ref, o_ref):
    s = p_ref[0] + p_ref[1]
    o_ref[...] = s[:, :9]


def _combine(parts):
    bn = 2000
    return pl.pallas_call(
        _combine_body,
        grid=(N_ATOMS // bn,),
        in_specs=[pl.BlockSpec((2, bn, 16), lambda i: (0, i, 0))],
        out_specs=pl.BlockSpec((bn, 9), lambda i: (i, 0)),
        out_shape=jax.ShapeDtypeStruct((N_ATOMS, 9), jnp.float32),
    )(parts)


def kernel(interatomic_vectors, centers, neighbors, species, sample_values,
           neighbor_embed, center_embed, W):
    del sample_values
    ce4 = center_embed.reshape(5, 8, 4)
    w4 = W.reshape(3, 8, 4)
    g = jnp.einsum("tj,sij,bij->tsbi", neighbor_embed, ce4, w4)
    g_flat = jnp.pad(g.reshape(-1), (0, 640 - 600)).astype(jnp.float32)

    vt3 = jnp.pad(interatomic_vectors.T,
                  ((0, 0), (0, E_PAD - N_EDGES))).reshape(3, VROWS, VCOL)
    cen2 = jnp.pad(centers, (0, E_PAD - N_EDGES)).reshape(E_PAD // IDXW, IDXW)
    nei2 = jnp.pad(neighbors, (0, E_PAD - N_EDGES)).reshape(E_PAD // IDXW, IDXW)
    sp4 = jnp.pad(species, (0, N_PAD - N_ATOMS)).reshape(N_PAD // 4, 4)
    sp_packed = (sp4[:, 0] | (sp4[:, 1] << 8) | (sp4[:, 2] << 16)
                 | (sp4[:, 3] << 24))

    p = _edge_feats(vt3)
    parts = _make_sc_scatter()(p, cen2, nei2, sp_packed, g_flat)
    return _combine(parts).reshape(N_ATOMS, 3, 3)


# speedup vs baseline: 1.2323x; 1.0643x over previous
import functools

import jax
import jax.numpy as jnp
import numpy as np
from jax import lax
from jax.experimental import pallas as pl
from jax.experimental.pallas import tpu as pltpu
from jax.experimental.pallas import tpu_sc as plsc

N_ATOMS = 50000
N_EDGES = 800000
CUTOFF = 5.0
WIDTH = 0.5

NC = 2
NS = 16
NW = NC * NS

E_PAD = 819200
EW = E_PAD // NW
CHUNK = 1024
IDXW = 128
IDXROWS = CHUNK // IDXW
NCHUNK = EW // CHUNK

VCOL = 512
VROWS = E_PAD // VCOL
CROWS = CHUNK // VCOL

N_PAD = 50176
ROWS_PER_TILE = N_PAD // NS

BROW = 8
PF = 12


def _edge_feat_body(v_ref, p_ref):
    vx = v_ref[0]
    vy = v_ref[1]
    vz = v_ref[2]
    r2 = vx * vx + vy * vy + vz * vz + 1e-12
    inv_r2 = 1.0 / r2
    r = jnp.sqrt(r2)
    theta = (jnp.pi / CUTOFF) * jnp.minimum(r, CUTOFF)
    u = theta - (jnp.pi / 2)
    u2 = u * u
    sin_u = u * (1.0 + u2 * (-1.6666667e-1 + u2 * (8.3333331e-3
            + u2 * (-1.9840874e-4 + u2 * 2.7525562e-6))))
    cos_u = 1.0 + u2 * (-0.5 + u2 * (4.1666668e-2 + u2 * (-1.3888889e-3
            + u2 * (2.4801587e-5 + u2 * -2.7557319e-7))))
    s1 = cos_u
    c1 = -sin_u
    two_c1 = 2.0 * c1
    sines = [s1, two_c1 * s1]
    for _ in range(6):
        sines.append(two_c1 * sines[-1] - sines[-2])
    s5 = sines[4]
    fc = jnp.where(r < CUTOFF - WIDTH, 1.0, s5 * s5)
    fc = jnp.where(r < CUTOFF, fc, 0.0)
    scale = inv_r2 * fc
    rows = jnp.stack(
        [s * scale for s in sines]
        + [vx, vy, vz, jnp.zeros((BROW, VCOL), jnp.float32)], axis=1)
    p_ref[...] = rows.reshape(BROW * PF, VCOL)


def _edge_feats(vt3):
    return pl.pallas_call(
        _edge_feat_body,
        grid=(VROWS // BROW,),
        in_specs=[pl.BlockSpec((3, BROW, VCOL), lambda i: (0, i, 0))],
        out_specs=pl.BlockSpec((BROW * PF, VCOL), lambda i: (i, 0)),
        out_shape=jax.ShapeDtypeStruct((VROWS * PF, VCOL), jnp.float32),
    )(vt3)


def _sc_body(p_hbm, cen_hbm, nei_hbm, sp_hbm, g_hbm, out_hbm,
             sp_v, g_v, p_v0, p_v1, cen_v0, cen_v1, nei_v0, nei_v1,
             stg_v, acc_sh, dsem0, dsem1, ssem):
    cid = lax.axis_index("c")
    sid = lax.axis_index("s")
    wid = sid * NC + cid

    pltpu.sync_copy(sp_hbm, sp_v)
    pltpu.sync_copy(g_hbm, g_v)

    zero16 = jnp.zeros((16,), jnp.float32)

    def _zero_body(i, _):
        stg_v[i, :] = zero16
        return 0

    lax.fori_loop(0, CHUNK, _zero_body, 0)

    quarter = ROWS_PER_TILE // 4
    r0 = sid * ROWS_PER_TILE
    for k in range(4):
        pltpu.sync_copy(stg_v.at[pl.ds(0, quarter)],
                        acc_sh.at[pl.ds(r0 + k * quarter, quarter)])
    plsc.subcore_barrier()

    lane = lax.iota(jnp.int32, 16)
    bufs = ((p_v0, cen_v0, nei_v0, dsem0), (p_v1, cen_v1, nei_v1, dsem1))

    def _fire(it, buf):
        p_v, cen_v, nei_v, sem = buf
        base = pl.multiple_of(wid * EW + it * CHUNK, CHUNK)
        pr0 = pl.multiple_of((base // VCOL) * PF, CROWS * PF)
        row0 = pl.multiple_of(base // IDXW, IDXROWS)
        pltpu.make_async_copy(
            p_hbm.at[pl.ds(pr0, CROWS * PF)], p_v, sem).start()
        pltpu.make_async_copy(
            cen_hbm.at[pl.ds(row0, IDXROWS)], cen_v, sem).start()
        pltpu.make_async_copy(
            nei_hbm.at[pl.ds(row0, IDXROWS)], nei_v, sem).start()

    def _wait(buf):
        p_v, cen_v, nei_v, sem = buf
        pltpu.make_async_copy(
            p_hbm.at[pl.ds(0, CROWS * PF)], p_v, sem).wait()
        pltpu.make_async_copy(
            cen_hbm.at[pl.ds(0, IDXROWS)], cen_v, sem).wait()
        pltpu.make_async_copy(
            nei_hbm.at[pl.ds(0, IDXROWS)], nei_v, sem).wait()

    def _process(buf):
        p_v, cen_v, nei_v, _ = buf

        def _a_body(a, _):
            def _r_body(r8, _):
                def _q_body(q, _):
                    colv = r8 * IDXW + q * 16
                    j = a * 4 + r8
                    off = q * 16
                    e0 = a * VCOL + colv
                    nvec = nei_v[j, pl.ds(off, 16)]
                    cvec = cen_v[j, pl.ds(off, 16)]
                    t_w = plsc.load_gather(
                        sp_v, [lax.shift_right_logical(nvec, 2)])
                    s_w = plsc.load_gather(
                        sp_v, [lax.shift_right_logical(cvec, 2)])
                    t_sp = lax.shift_right_logical(
                        t_w, lax.shift_left(nvec & 3, 3)) & 7
                    s_sp = lax.shift_right_logical(
                        s_w, lax.shift_left(cvec & 3, 3)) & 7
                    u24 = (t_sp * 5 + s_sp) * 24
                    col = pl.ds(colv, 16)
                    a16 = a * PF
                    rfc = [p_v[a16 + i, col] for i in range(8)]
                    ys = []
                    for b in range(3):
                        acc = rfc[0] * plsc.load_gather(g_v, [u24 + (b * 8)])
                        for i in range(1, 8):
                            acc = acc + rfc[i] * plsc.load_gather(
                                g_v, [u24 + (b * 8 + i)])
                        ys.append(acc)
                    rows = e0 + lane
                    for m in range(3):
                        d = p_v[a16 + 8 + m, col]
                        for b in range(3):
                            plsc.store_scatter(
                                stg_v,
                                [rows, jnp.full((16,), m * 3 + b, jnp.int32)],
                                d * ys[b])
                    return 0

                lax.fori_loop(0, IDXW // 16, _q_body, 0)
                return 0

            lax.fori_loop(0, VCOL // IDXW, _r_body, 0)
            return 0

        lax.fori_loop(0, CROWS, _a_body, 0)

        descs = []
        for j in range(IDXROWS):
            d = pltpu.make_async_copy(stg_v.at[pl.ds(j * IDXW, IDXW)],
                                      acc_sh.at[cen_v.at[j]], ssem)
            d.start(add=True)
            descs.append(d)
        for d in descs:
            d.wait()

    _fire(0, bufs[0])

    def _chunk_body(it, _):
        @pl.when(lax.rem(it, 2) == 0)
        def _():
            _wait(bufs[0])

            @pl.when(it + 1 < NCHUNK)
            def _():
                _fire(it + 1, bufs[1])

            _process(bufs[0])

        @pl.when(lax.rem(it, 2) == 1)
        def _():
            _wait(bufs[1])

            @pl.when(it + 1 < NCHUNK)
            def _():
                _fire(it + 1, bufs[0])

            _process(bufs[1])

        return 0

    lax.fori_loop(0, NCHUNK, _chunk_body, 0)

    plsc.subcore_barrier()
    pltpu.sync_copy(acc_sh.at[pl.ds(r0, ROWS_PER_TILE)],
                    out_hbm.at[cid, pl.ds(r0, ROWS_PER_TILE)])


@functools.cache
def _make_sc_scatter():
    return pl.kernel(
        _sc_body,
        out_type=jax.ShapeDtypeStruct((NC, N_PAD, 16), jnp.float32),
        mesh=plsc.VectorSubcoreMesh(core_axis_name="c", subcore_axis_name="s",
                                    num_cores=NC, num_subcores=NS),
        scratch_types=[
            pltpu.VMEM((N_PAD // 4,), jnp.int32),
            pltpu.VMEM((640,), jnp.float32),
            pltpu.VMEM((CROWS * PF, VCOL), jnp.float32),
            pltpu.VMEM((CROWS * PF, VCOL), jnp.float32),
            pltpu.VMEM((IDXROWS, IDXW), jnp.int32),
            pltpu.VMEM((IDXROWS, IDXW), jnp.int32),
            pltpu.VMEM((IDXROWS, IDXW), jnp.int32),
            pltpu.VMEM((IDXROWS, IDXW), jnp.int32),
            pltpu.VMEM((CHUNK, 16), jnp.float32),
            pltpu.VMEM_SHARED((N_PAD, 16), jnp.float32),
            pltpu.SemaphoreType.DMA,
            pltpu.SemaphoreType.DMA,
            pltpu.SemaphoreType.DMA,
        ],
        compiler_params=pltpu.CompilerParams(needs_layout_passes=False,
                                             use_tc_tiling_on_sc=False),
    )


def _combine_body(p_ref, o_ref):
    o_ref[...] = p_ref[0] + p_ref[1]


def _combine(parts):
    rows = N_PAD // 8
    bn8 = rows // 16
    flat = parts.reshape(NC, rows, 128)
    out = pl.pallas_call(
        _combine_body,
        grid=(16,),
        in_specs=[pl.BlockSpec((2, bn8, 128), lambda i: (0, i, 0))],
        out_specs=pl.BlockSpec((bn8, 128), lambda i: (i, 0)),
        out_shape=jax.ShapeDtypeStruct((rows, 128), jnp.float32),
    )(flat)
    return out.reshape(N_PAD, 16)


def kernel(interatomic_vectors, centers, neighbors, species, sample_values,
           neighbor_embed, center_embed, W):
    del sample_values
    ce4 = center_embed.reshape(5, 8, 4)
    w4 = W.reshape(3, 8, 4)
    g = jnp.einsum("tj,sij,bij->tsbi", neighbor_embed, ce4, w4)
    g_flat = jnp.pad(g.reshape(-1), (0, 640 - 600)).astype(jnp.float32)

    vt3 = jnp.pad(interatomic_vectors.T,
                  ((0, 0), (0, E_PAD - N_EDGES))).reshape(3, VROWS, VCOL)
    cen2 = jnp.pad(centers, (0, E_PAD - N_EDGES)).reshape(E_PAD // IDXW, IDXW)
    nei2 = jnp.pad(neighbors, (0, E_PAD - N_EDGES)).reshape(E_PAD // IDXW, IDXW)
    sp4 = jnp.pad(species, (0, N_PAD - N_ATOMS)).reshape(N_PAD // 4, 4)
    sp_packed = (sp4[:, 0] | (sp4[:, 1] << 8) | (sp4[:, 2] << 16)
                 | (sp4[:, 3] << 24))

    p = _edge_feats(vt3)
    parts = _make_sc_scatter()(p, cen2, nei2, sp_packed, g_flat)
    return _combine(parts)[:N_ATOMS, :9].reshape(N_ATOMS, 3, 3)


# speedup vs baseline: 1.2323x; 1.0000x over previous
import functools

import jax
import jax.numpy as jnp
import numpy as np
from jax import lax
from jax.experimental import pallas as pl
from jax.experimental.pallas import tpu as pltpu
from jax.experimental.pallas import tpu_sc as plsc

N_ATOMS = 50000
N_EDGES = 800000
CUTOFF = 5.0
WIDTH = 0.5

NC = 2
NS = 16
NW = NC * NS

E_PAD = 819200
EW = E_PAD // NW
CHUNK = 1024
IDXW = 128
IDXROWS = CHUNK // IDXW
NCHUNK = EW // CHUNK

VCOL = 512
VROWS = E_PAD // VCOL
CROWS = CHUNK // VCOL

N_PAD = 50176
ROWS_PER_TILE = N_PAD // NS

BROW = 8
PF = 12


def _edge_feat_body(v_ref, p_ref):
    vx = v_ref[0]
    vy = v_ref[1]
    vz = v_ref[2]
    r2 = vx * vx + vy * vy + vz * vz + 1e-12
    inv_r2 = 1.0 / r2
    r = jnp.sqrt(r2)
    theta = (jnp.pi / CUTOFF) * jnp.minimum(r, CUTOFF)
    u = theta - (jnp.pi / 2)
    u2 = u * u
    sin_u = u * (1.0 + u2 * (-1.6666667e-1 + u2 * (8.3333331e-3
            + u2 * (-1.9840874e-4 + u2 * 2.7525562e-6))))
    cos_u = 1.0 + u2 * (-0.5 + u2 * (4.1666668e-2 + u2 * (-1.3888889e-3
            + u2 * (2.4801587e-5 + u2 * -2.7557319e-7))))
    s1 = cos_u
    c1 = -sin_u
    two_c1 = 2.0 * c1
    sines = [s1, two_c1 * s1]
    for _ in range(6):
        sines.append(two_c1 * sines[-1] - sines[-2])
    s5 = sines[4]
    fc = jnp.where(r < CUTOFF - WIDTH, 1.0, s5 * s5)
    fc = jnp.where(r < CUTOFF, fc, 0.0)
    scale = inv_r2 * fc
    rows = jnp.stack(
        [s * scale for s in sines]
        + [vx, vy, vz, jnp.zeros((BROW, VCOL), jnp.float32)], axis=1)
    p_ref[...] = rows.reshape(BROW * PF, VCOL)


def _edge_feats(vt3):
    return pl.pallas_call(
        _edge_feat_body,
        grid=(VROWS // BROW,),
        in_specs=[pl.BlockSpec((3, BROW, VCOL), lambda i: (0, i, 0))],
        out_specs=pl.BlockSpec((BROW * PF, VCOL), lambda i: (i, 0)),
        out_shape=jax.ShapeDtypeStruct((VROWS * PF, VCOL), jnp.float32),
    )(vt3)


def _sc_body(p_hbm, cen_hbm, nei_hbm, sp_hbm, g_hbm, out_hbm,
             sp_v, g_v, p_v0, p_v1, cen_v0, cen_v1, nei_v0, nei_v1,
             stg_v, stg_w, acc_sh, dsem0, dsem1, ssem0, ssem1):
    cid = lax.axis_index("c")
    sid = lax.axis_index("s")
    wid = sid * NC + cid

    pltpu.sync_copy(sp_hbm, sp_v)
    pltpu.sync_copy(g_hbm, g_v)

    zero16 = jnp.zeros((16,), jnp.float32)

    def _zero_body(i, _):
        stg_v[i, :] = zero16
        stg_w[i, :] = zero16
        return 0

    lax.fori_loop(0, CHUNK, _zero_body, 0)

    quarter = ROWS_PER_TILE // 4
    r0 = sid * ROWS_PER_TILE
    for k in range(4):
        pltpu.sync_copy(stg_v.at[pl.ds(0, quarter)],
                        acc_sh.at[pl.ds(r0 + k * quarter, quarter)])
    plsc.subcore_barrier()

    lane = lax.iota(jnp.int32, 16)
    bufs = ((p_v0, cen_v0, nei_v0, dsem0), (p_v1, cen_v1, nei_v1, dsem1))

    def _fire(it, buf):
        p_v, cen_v, nei_v, sem = buf
        base = pl.multiple_of(wid * EW + it * CHUNK, CHUNK)
        pr0 = pl.multiple_of((base // VCOL) * PF, CROWS * PF)
        row0 = pl.multiple_of(base // IDXW, IDXROWS)
        pltpu.make_async_copy(
            p_hbm.at[pl.ds(pr0, CROWS * PF)], p_v, sem).start()
        pltpu.make_async_copy(
            cen_hbm.at[pl.ds(row0, IDXROWS)], cen_v, sem).start()
        pltpu.make_async_copy(
            nei_hbm.at[pl.ds(row0, IDXROWS)], nei_v, sem).start()

    def _wait(buf):
        p_v, cen_v, nei_v, sem = buf
        pltpu.make_async_copy(
            p_hbm.at[pl.ds(0, CROWS * PF)], p_v, sem).wait()
        pltpu.make_async_copy(
            cen_hbm.at[pl.ds(0, IDXROWS)], cen_v, sem).wait()
        pltpu.make_async_copy(
            nei_hbm.at[pl.ds(0, IDXROWS)], nei_v, sem).wait()

    def _drain(stg, cen_v, ssem):
        for j in range(IDXROWS):
            pltpu.make_async_copy(stg.at[pl.ds(j * IDXW, IDXW)],
                                  acc_sh.at[cen_v.at[j]], ssem).wait()

    def _process(buf, stg, ssem):
        p_v, cen_v, nei_v, _ = buf

        def _a_body(a, _):
            def _r_body(r8, _):
                def _q_body(q, _):
                    colv = r8 * IDXW + q * 16
                    j = a * 4 + r8
                    off = q * 16
                    e0 = a * VCOL + colv
                    nvec = nei_v[j, pl.ds(off, 16)]
                    cvec = cen_v[j, pl.ds(off, 16)]
                    t_w = plsc.load_gather(
                        sp_v, [lax.shift_right_logical(nvec, 2)])
                    s_w = plsc.load_gather(
                        sp_v, [lax.shift_right_logical(cvec, 2)])
                    t_sp = lax.shift_right_logical(
                        t_w, lax.shift_left(nvec & 3, 3)) & 7
                    s_sp = lax.shift_right_logical(
                        s_w, lax.shift_left(cvec & 3, 3)) & 7
                    u24 = (t_sp * 5 + s_sp) * 24
                    col = pl.ds(colv, 16)
                    a16 = a * PF
                    rfc = [p_v[a16 + i, col] for i in range(8)]
                    ys = []
                    for b in range(3):
                        acc = rfc[0] * plsc.load_gather(g_v, [u24 + (b * 8)])
                        for i in range(1, 8):
                            acc = acc + rfc[i] * plsc.load_gather(
                                g_v, [u24 + (b * 8 + i)])
                        ys.append(acc)
                    rows = e0 + lane
                    for m in range(3):
                        d = p_v[a16 + 8 + m, col]
                        for b in range(3):
                            plsc.store_scatter(
                                stg,
                                [rows, jnp.full((16,), m * 3 + b, jnp.int32)],
                                d * ys[b])
                    return 0

                lax.fori_loop(0, IDXW // 16, _q_body, 0)
                return 0

            lax.fori_loop(0, VCOL // IDXW, _r_body, 0)
            return 0

        lax.fori_loop(0, CROWS, _a_body, 0)

        for j in range(IDXROWS):
            pltpu.make_async_copy(stg.at[pl.ds(j * IDXW, IDXW)],
                                  acc_sh.at[cen_v.at[j]], ssem).start(add=True)

    _fire(0, bufs[0])
    stgs = (stg_v, stg_w)
    ssems = (ssem0, ssem1)

    def _chunk_body(it, _):
        @pl.when(lax.rem(it, 2) == 0)
        def _():
            _wait(bufs[0])

            @pl.when(it >= 1)
            def _():
                _drain(stgs[1], bufs[1][1], ssems[1])

            @pl.when(it + 1 < NCHUNK)
            def _():
                _fire(it + 1, bufs[1])

            _process(bufs[0], stgs[0], ssems[0])

        @pl.when(lax.rem(it, 2) == 1)
        def _():
            _wait(bufs[1])

            @pl.when(it >= 1)
            def _():
                _drain(stgs[0], bufs[0][1], ssems[0])

            @pl.when(it + 1 < NCHUNK)
            def _():
                _fire(it + 1, bufs[0])

            _process(bufs[1], stgs[1], ssems[1])

        return 0

    lax.fori_loop(0, NCHUNK, _chunk_body, 0)
    _drain(stgs[(NCHUNK - 1) % 2], bufs[(NCHUNK - 1) % 2][1],
           ssems[(NCHUNK - 1) % 2])

    plsc.subcore_barrier()
    pltpu.sync_copy(acc_sh.at[pl.ds(r0, ROWS_PER_TILE)],
                    out_hbm.at[cid, pl.ds(r0, ROWS_PER_TILE)])


@functools.cache
def _make_sc_scatter():
    return pl.kernel(
        _sc_body,
        out_type=jax.ShapeDtypeStruct((NC, N_PAD, 16), jnp.float32),
        mesh=plsc.VectorSubcoreMesh(core_axis_name="c", subcore_axis_name="s",
                                    num_cores=NC, num_subcores=NS),
        scratch_types=[
            pltpu.VMEM((N_PAD // 4,), jnp.int32),
            pltpu.VMEM((640,), jnp.float32),
            pltpu.VMEM((CROWS * PF, VCOL), jnp.float32),
            pltpu.VMEM((CROWS * PF, VCOL), jnp.float32),
            pltpu.VMEM((IDXROWS, IDXW), jnp.int32),
            pltpu.VMEM((IDXROWS, IDXW), jnp.int32),
            pltpu.VMEM((IDXROWS, IDXW), jnp.int32),
            pltpu.VMEM((IDXROWS, IDXW), jnp.int32),
            pltpu.VMEM((CHUNK, 16), jnp.float32),
            pltpu.VMEM((CHUNK, 16), jnp.float32),
            pltpu.VMEM_SHARED((N_PAD, 16), jnp.float32),
            pltpu.SemaphoreType.DMA,
            pltpu.SemaphoreType.DMA,
            pltpu.SemaphoreType.DMA,
            pltpu.SemaphoreType.DMA,
        ],
        compiler_params=pltpu.CompilerParams(needs_layout_passes=False,
                                             use_tc_tiling_on_sc=False),
    )


def _combine_body(p_ref, o_ref):
    o_ref[...] = p_ref[0] + p_ref[1]


def _combine(parts):
    rows = N_PAD // 8
    bn8 = rows // 16
    flat = parts.reshape(NC, rows, 128)
    out = pl.pallas_call(
        _combine_body,
        grid=(16,),
        in_specs=[pl.BlockSpec((2, bn8, 128), lambda i: (0, i, 0))],
        out_specs=pl.BlockSpec((bn8, 128), lambda i: (i, 0)),
        out_shape=jax.ShapeDtypeStruct((rows, 128), jnp.float32),
    )(flat)
    return out.reshape(N_PAD, 16)


def kernel(interatomic_vectors, centers, neighbors, species, sample_values,
           neighbor_embed, center_embed, W):
    del sample_values
    ce4 = center_embed.reshape(5, 8, 4)
    w4 = W.reshape(3, 8, 4)
    g = jnp.einsum("tj,sij,bij->tsbi", neighbor_embed, ce4, w4)
    g_flat = jnp.pad(g.reshape(-1), (0, 640 - 600)).astype(jnp.float32)

    vt3 = jnp.pad(interatomic_vectors.T,
                  ((0, 0), (0, E_PAD - N_EDGES))).reshape(3, VROWS, VCOL)
    p = _edge_feats(vt3)

    cen2 = jnp.pad(centers, (0, E_PAD - N_EDGES)).reshape(E_PAD // IDXW, IDXW)
    nei2 = jnp.pad(neighbors, (0, E_PAD - N_EDGES)).reshape(E_PAD // IDXW, IDXW)
    sp4 = jnp.pad(species, (0, N_PAD - N_ATOMS)).reshape(N_PAD // 4, 4)
    sp_packed = (sp4[:, 0] | (sp4[:, 1] << 8) | (sp4[:, 2] << 16)
                 | (sp4[:, 3] << 24))

    parts = _make_sc_scatter()(p, cen2, nei2, sp_packed, g_flat)
    return _combine(parts)[:N_ATOMS, :9].reshape(N_ATOMS, 3, 3)


# speedup vs baseline: 1.4722x; 1.1946x over previous
import functools

import jax
import jax.numpy as jnp
import numpy as np
from jax import lax
from jax.experimental import pallas as pl
from jax.experimental.pallas import tpu as pltpu
from jax.experimental.pallas import tpu_sc as plsc

N_ATOMS = 50000
N_EDGES = 800000
CUTOFF = 5.0
WIDTH = 0.5

NC = 2
NS = 16
NW = NC * NS

E_PAD = 819200
EW = E_PAD // NW
CHUNK = 1024
IDXW = 128
IDXROWS = CHUNK // IDXW
NCHUNK = EW // CHUNK

VCOL = 512
VROWS = E_PAD // VCOL
CROWS = CHUNK // VCOL

N_PAD = 50176
ROWS_PER_TILE = N_PAD // NS

BROW = 32
PF = 12


def _edge_feat_body(v_ref, p_ref):
    vx = v_ref[0]
    vy = v_ref[1]
    vz = v_ref[2]
    r2 = vx * vx + vy * vy + vz * vz + 1e-12
    inv_r2 = 1.0 / r2
    r = jnp.sqrt(r2)
    theta = (jnp.pi / CUTOFF) * jnp.minimum(r, CUTOFF)
    u = theta - (jnp.pi / 2)
    u2 = u * u
    sin_u = u * (1.0 + u2 * (-1.6666667e-1 + u2 * (8.3333331e-3
            + u2 * (-1.9840874e-4 + u2 * 2.7525562e-6))))
    cos_u = 1.0 + u2 * (-0.5 + u2 * (4.1666668e-2 + u2 * (-1.3888889e-3
            + u2 * (2.4801587e-5 + u2 * -2.7557319e-7))))
    s1 = cos_u
    c1 = -sin_u
    two_c1 = 2.0 * c1
    sines = [s1, two_c1 * s1]
    for _ in range(6):
        sines.append(two_c1 * sines[-1] - sines[-2])
    s5 = sines[4]
    fc = jnp.where(r < CUTOFF - WIDTH, 1.0, s5 * s5)
    fc = jnp.where(r < CUTOFF, fc, 0.0)
    scale = inv_r2 * fc
    rows = jnp.stack(
        [s * scale for s in sines]
        + [vx, vy, vz, jnp.zeros((BROW, VCOL), jnp.float32)], axis=1)
    p_ref[...] = rows.reshape(BROW * PF, VCOL)


def _edge_feats(vt3):
    return pl.pallas_call(
        _edge_feat_body,
        grid=(VROWS // BROW,),
        in_specs=[pl.BlockSpec((3, BROW, VCOL), lambda i: (0, i, 0))],
        out_specs=pl.BlockSpec((BROW * PF, VCOL), lambda i: (i, 0)),
        out_shape=jax.ShapeDtypeStruct((VROWS * PF, VCOL), jnp.float32),
    )(vt3)


def _sc_body(p_hbm, cen_hbm, nei_hbm, sp_hbm, g_hbm, out_hbm,
             sp_v, g_v, p_v0, p_v1, cen_v0, cen_v1, nei_v0, nei_v1,
             stg_v, stg_w, acc_sh, dsem0, dsem1, ssem0, ssem1):
    cid = lax.axis_index("c")
    sid = lax.axis_index("s")
    wid = sid * NC + cid

    pltpu.sync_copy(sp_hbm, sp_v)
    pltpu.sync_copy(g_hbm, g_v)

    zero16 = jnp.zeros((16,), jnp.float32)

    def _zero_body(i, _):
        stg_v[i, :] = zero16
        stg_w[i, :] = zero16
        return 0

    lax.fori_loop(0, CHUNK, _zero_body, 0)

    quarter = ROWS_PER_TILE // 4
    r0 = sid * ROWS_PER_TILE
    for k in range(4):
        pltpu.sync_copy(stg_v.at[pl.ds(0, quarter)],
                        acc_sh.at[pl.ds(r0 + k * quarter, quarter)])
    plsc.subcore_barrier()

    lane = lax.iota(jnp.int32, 16)
    bufs = ((p_v0, cen_v0, nei_v0, dsem0), (p_v1, cen_v1, nei_v1, dsem1))

    def _fire(it, buf):
        p_v, cen_v, nei_v, sem = buf
        base = pl.multiple_of(wid * EW + it * CHUNK, CHUNK)
        pr0 = pl.multiple_of((base // VCOL) * PF, CROWS * PF)
        row0 = pl.multiple_of(base // IDXW, IDXROWS)
        pltpu.make_async_copy(
            p_hbm.at[pl.ds(pr0, CROWS * PF)], p_v, sem).start()
        pltpu.make_async_copy(
            cen_hbm.at[pl.ds(row0, IDXROWS)], cen_v, sem).start()
        pltpu.make_async_copy(
            nei_hbm.at[pl.ds(row0, IDXROWS)], nei_v, sem).start()

    def _wait(buf):
        p_v, cen_v, nei_v, sem = buf
        pltpu.make_async_copy(
            p_hbm.at[pl.ds(0, CROWS * PF)], p_v, sem).wait()
        pltpu.make_async_copy(
            cen_hbm.at[pl.ds(0, IDXROWS)], cen_v, sem).wait()
        pltpu.make_async_copy(
            nei_hbm.at[pl.ds(0, IDXROWS)], nei_v, sem).wait()

    def _drain(stg, cen_v, ssem):
        for j in range(IDXROWS):
            pltpu.make_async_copy(stg.at[pl.ds(j * IDXW, IDXW)],
                                  acc_sh.at[cen_v.at[j]], ssem).wait()

    def _process(buf, stg, ssem):
        p_v, cen_v, nei_v, _ = buf

        def _a_body(a, _):
            def _r_body(r8, _):
                def _q_body(q, _):
                    colv = r8 * IDXW + q * 16
                    j = a * 4 + r8
                    off = q * 16
                    e0 = a * VCOL + colv
                    nvec = nei_v[j, pl.ds(off, 16)]
                    cvec = cen_v[j, pl.ds(off, 16)]
                    t_w = plsc.load_gather(
                        sp_v, [lax.shift_right_logical(nvec, 2)])
                    s_w = plsc.load_gather(
                        sp_v, [lax.shift_right_logical(cvec, 2)])
                    t_sp = lax.shift_right_logical(
                        t_w, lax.shift_left(nvec & 3, 3)) & 7
                    s_sp = lax.shift_right_logical(
                        s_w, lax.shift_left(cvec & 3, 3)) & 7
                    u24 = (t_sp * 5 + s_sp) * 24
                    col = pl.ds(colv, 16)
                    a16 = a * PF
                    rfc = [p_v[a16 + i, col] for i in range(8)]
                    ys = []
                    for b in range(3):
                        acc = rfc[0] * plsc.load_gather(g_v, [u24 + (b * 8)])
                        for i in range(1, 8):
                            acc = acc + rfc[i] * plsc.load_gather(
                                g_v, [u24 + (b * 8 + i)])
                        ys.append(acc)
                    rows = e0 + lane
                    for m in range(3):
                        d = p_v[a16 + 8 + m, col]
                        for b in range(3):
                            plsc.store_scatter(
                                stg,
                                [rows, jnp.full((16,), m * 3 + b, jnp.int32)],
                                d * ys[b])
                    return 0

                lax.fori_loop(0, IDXW // 16, _q_body, 0)
                return 0

            lax.fori_loop(0, VCOL // IDXW, _r_body, 0)
            return 0

        lax.fori_loop(0, CROWS, _a_body, 0)

        for j in range(IDXROWS):
            pltpu.make_async_copy(stg.at[pl.ds(j * IDXW, IDXW)],
                                  acc_sh.at[cen_v.at[j]], ssem).start(add=True)

    _fire(0, bufs[0])
    stgs = (stg_v, stg_w)
    ssems = (ssem0, ssem1)

    def _chunk_body(it, _):
        @pl.when(lax.rem(it, 2) == 0)
        def _():
            _wait(bufs[0])

            @pl.when(it >= 1)
            def _():
                _drain(stgs[1], bufs[1][1], ssems[1])

            @pl.when(it + 1 < NCHUNK)
            def _():
                _fire(it + 1, bufs[1])

            _process(bufs[0], stgs[0], ssems[0])

        @pl.when(lax.rem(it, 2) == 1)
        def _():
            _wait(bufs[1])

            @pl.when(it >= 1)
            def _():
                _drain(stgs[0], bufs[0][1], ssems[0])

            @pl.when(it + 1 < NCHUNK)
            def _():
                _fire(it + 1, bufs[0])

            _process(bufs[1], stgs[1], ssems[1])

        return 0

    lax.fori_loop(0, NCHUNK, _chunk_body, 0)
    _drain(stgs[(NCHUNK - 1) % 2], bufs[(NCHUNK - 1) % 2][1],
           ssems[(NCHUNK - 1) % 2])

    plsc.subcore_barrier()
    pltpu.sync_copy(acc_sh.at[pl.ds(r0, ROWS_PER_TILE)],
                    out_hbm.at[cid, pl.ds(r0, ROWS_PER_TILE)])


@functools.cache
def _make_sc_scatter():
    return pl.kernel(
        _sc_body,
        out_type=jax.ShapeDtypeStruct((NC, N_PAD, 16), jnp.float32),
        mesh=plsc.VectorSubcoreMesh(core_axis_name="c", subcore_axis_name="s",
                                    num_cores=NC, num_subcores=NS),
        scratch_types=[
            pltpu.VMEM((N_PAD // 4,), jnp.int32),
            pltpu.VMEM((640,), jnp.float32),
            pltpu.VMEM((CROWS * PF, VCOL), jnp.float32),
            pltpu.VMEM((CROWS * PF, VCOL), jnp.float32),
            pltpu.VMEM((IDXROWS, IDXW), jnp.int32),
            pltpu.VMEM((IDXROWS, IDXW), jnp.int32),
            pltpu.VMEM((IDXROWS, IDXW), jnp.int32),
            pltpu.VMEM((IDXROWS, IDXW), jnp.int32),
            pltpu.VMEM((CHUNK, 16), jnp.float32),
            pltpu.VMEM((CHUNK, 16), jnp.float32),
            pltpu.VMEM_SHARED((N_PAD, 16), jnp.float32),
            pltpu.SemaphoreType.DMA,
            pltpu.SemaphoreType.DMA,
            pltpu.SemaphoreType.DMA,
            pltpu.SemaphoreType.DMA,
        ],
        compiler_params=pltpu.CompilerParams(needs_layout_passes=False,
                                             use_tc_tiling_on_sc=False),
    )


def _combine_body(p_ref, o_ref):
    o_ref[...] = p_ref[0] + p_ref[1]


def _combine(parts):
    rows = N_PAD // 8
    bn8 = rows // 16
    flat = parts.reshape(NC, rows, 128)
    out = pl.pallas_call(
        _combine_body,
        grid=(16,),
        in_specs=[pl.BlockSpec((2, bn8, 128), lambda i: (0, i, 0))],
        out_specs=pl.BlockSpec((bn8, 128), lambda i: (i, 0)),
        out_shape=jax.ShapeDtypeStruct((rows, 128), jnp.float32),
    )(flat)
    return out.reshape(N_PAD, 16)


def kernel(interatomic_vectors, centers, neighbors, species, sample_values,
           neighbor_embed, center_embed, W):
    del sample_values
    ce4 = center_embed.reshape(5, 8, 4)
    w4 = W.reshape(3, 8, 4)
    g = jnp.einsum("tj,sij,bij->tsbi", neighbor_embed, ce4, w4)
    g_flat = jnp.pad(g.reshape(-1), (0, 640 - 600)).astype(jnp.float32)

    vt3 = jnp.pad(interatomic_vectors.T,
                  ((0, 0), (0, E_PAD - N_EDGES))).reshape(3, VROWS, VCOL)
    p = _edge_feats(vt3)

    cen2 = jnp.pad(centers, (0, E_PAD - N_EDGES)).reshape(E_PAD // IDXW, IDXW)
    nei2 = jnp.pad(neighbors, (0, E_PAD - N_EDGES)).reshape(E_PAD // IDXW, IDXW)
    sp4 = jnp.pad(species, (0, N_PAD - N_ATOMS)).reshape(N_PAD // 4, 4)
    sp_packed = (sp4[:, 0] | (sp4[:, 1] << 8) | (sp4[:, 2] << 16)
                 | (sp4[:, 3] << 24))

    parts = _make_sc_scatter()(p, cen2, nei2, sp_packed, g_flat)
    return _combine(parts)[:N_ATOMS, :9].reshape(N_ATOMS, 3, 3)


# speedup vs baseline: 1.5205x; 1.0328x over previous
import functools

import jax
import jax.numpy as jnp
import numpy as np
from jax import lax
from jax.experimental import pallas as pl
from jax.experimental.pallas import tpu as pltpu
from jax.experimental.pallas import tpu_sc as plsc

N_ATOMS = 50000
N_EDGES = 800000
CUTOFF = 5.0
WIDTH = 0.5

NC = 2
NS = 16
NW = NC * NS

E_PAD = 819200
EW = E_PAD // NW
CHUNK = 1024
IDXW = 128
IDXROWS = CHUNK // IDXW
NCHUNK = EW // CHUNK

VCOL = 512
VROWS = E_PAD // VCOL
CROWS = CHUNK // VCOL

N_PAD = 50176
ROWS_PER_TILE = N_PAD // NS

BROW = 64
PF = 12


def _edge_feat_body(v_ref, p_ref):
    vx = v_ref[0]
    vy = v_ref[1]
    vz = v_ref[2]
    r2 = vx * vx + vy * vy + vz * vz + 1e-12
    inv_r2 = 1.0 / r2
    r = jnp.sqrt(r2)
    theta = (jnp.pi / CUTOFF) * jnp.minimum(r, CUTOFF)
    u = theta - (jnp.pi / 2)
    u2 = u * u
    sin_u = u * (1.0 + u2 * (-1.6666667e-1 + u2 * (8.3333331e-3
            + u2 * (-1.9840874e-4 + u2 * 2.7525562e-6))))
    cos_u = 1.0 + u2 * (-0.5 + u2 * (4.1666668e-2 + u2 * (-1.3888889e-3
            + u2 * (2.4801587e-5 + u2 * -2.7557319e-7))))
    s1 = cos_u
    c1 = -sin_u
    two_c1 = 2.0 * c1
    sines = [s1, two_c1 * s1]
    for _ in range(6):
        sines.append(two_c1 * sines[-1] - sines[-2])
    s5 = sines[4]
    fc = jnp.where(r < CUTOFF - WIDTH, 1.0, s5 * s5)
    fc = jnp.where(r < CUTOFF, fc, 0.0)
    scale = inv_r2 * fc
    rows = jnp.stack(
        [s * scale for s in sines]
        + [vx, vy, vz, jnp.zeros((BROW, VCOL), jnp.float32)], axis=1)
    p_ref[...] = rows.reshape(BROW * PF, VCOL)


def _edge_feats(vt3):
    return pl.pallas_call(
        _edge_feat_body,
        grid=(VROWS // BROW,),
        in_specs=[pl.BlockSpec((3, BROW, VCOL), lambda i: (0, i, 0))],
        out_specs=pl.BlockSpec((BROW * PF, VCOL), lambda i: (i, 0)),
        out_shape=jax.ShapeDtypeStruct((VROWS * PF, VCOL), jnp.float32),
    )(vt3)


def _sc_body(p_hbm, cen_hbm, nei_hbm, sp_hbm, g_hbm, out_hbm,
             sp_v, g_v, p_v0, p_v1, cen_v0, cen_v1, nei_v0, nei_v1,
             stg_v, stg_w, acc_sh, dsem0, dsem1, ssem0, ssem1):
    cid = lax.axis_index("c")
    sid = lax.axis_index("s")
    wid = sid * NC + cid

    pltpu.sync_copy(sp_hbm, sp_v)
    pltpu.sync_copy(g_hbm, g_v)

    zero16 = jnp.zeros((16,), jnp.float32)

    def _zero_body(i, _):
        stg_v[i, :] = zero16
        stg_w[i, :] = zero16
        return 0

    lax.fori_loop(0, CHUNK, _zero_body, 0)

    quarter = ROWS_PER_TILE // 4
    r0 = sid * ROWS_PER_TILE
    for k in range(4):
        pltpu.sync_copy(stg_v.at[pl.ds(0, quarter)],
                        acc_sh.at[pl.ds(r0 + k * quarter, quarter)])
    plsc.subcore_barrier()

    lane = lax.iota(jnp.int32, 16)
    bufs = ((p_v0, cen_v0, nei_v0, dsem0), (p_v1, cen_v1, nei_v1, dsem1))

    def _fire(it, buf):
        p_v, cen_v, nei_v, sem = buf
        base = pl.multiple_of(wid * EW + it * CHUNK, CHUNK)
        pr0 = pl.multiple_of((base // VCOL) * PF, CROWS * PF)
        row0 = pl.multiple_of(base // IDXW, IDXROWS)
        pltpu.make_async_copy(
            p_hbm.at[pl.ds(pr0, CROWS * PF)], p_v, sem).start()
        pltpu.make_async_copy(
            cen_hbm.at[pl.ds(row0, IDXROWS)], cen_v, sem).start()
        pltpu.make_async_copy(
            nei_hbm.at[pl.ds(row0, IDXROWS)], nei_v, sem).start()

    def _wait(buf):
        p_v, cen_v, nei_v, sem = buf
        pltpu.make_async_copy(
            p_hbm.at[pl.ds(0, CROWS * PF)], p_v, sem).wait()
        pltpu.make_async_copy(
            cen_hbm.at[pl.ds(0, IDXROWS)], cen_v, sem).wait()
        pltpu.make_async_copy(
            nei_hbm.at[pl.ds(0, IDXROWS)], nei_v, sem).wait()

    def _drain(stg, cen_v, ssem):
        for j in range(IDXROWS):
            pltpu.make_async_copy(stg.at[pl.ds(j * IDXW, IDXW)],
                                  acc_sh.at[cen_v.at[j]], ssem).wait()

    def _process(buf, stg, ssem):
        p_v, cen_v, nei_v, _ = buf

        def _a_body(a, _):
            def _r_body(r8, _):
                def _q_body(q, _):
                    colv = r8 * IDXW + q * 16
                    j = a * 4 + r8
                    off = q * 16
                    e0 = a * VCOL + colv
                    nvec = nei_v[j, pl.ds(off, 16)]
                    cvec = cen_v[j, pl.ds(off, 16)]
                    t_w = plsc.load_gather(
                        sp_v, [lax.shift_right_logical(nvec, 2)])
                    s_w = plsc.load_gather(
                        sp_v, [lax.shift_right_logical(cvec, 2)])
                    t_sp = lax.shift_right_logical(
                        t_w, lax.shift_left(nvec & 3, 3)) & 7
                    s_sp = lax.shift_right_logical(
                        s_w, lax.shift_left(cvec & 3, 3)) & 7
                    u24 = (t_sp * 5 + s_sp) * 24
                    col = pl.ds(colv, 16)
                    a16 = a * PF
                    rfc = [p_v[a16 + i, col] for i in range(8)]
                    ys = []
                    for b in range(3):
                        acc = rfc[0] * plsc.load_gather(g_v, [u24 + (b * 8)])
                        for i in range(1, 8):
                            acc = acc + rfc[i] * plsc.load_gather(
                                g_v, [u24 + (b * 8 + i)])
                        ys.append(acc)
                    rows = e0 + lane
                    for m in range(3):
                        d = p_v[a16 + 8 + m, col]
                        for b in range(3):
                            plsc.store_scatter(
                                stg,
                                [rows, jnp.full((16,), m * 3 + b, jnp.int32)],
                                d * ys[b])
                    return 0

                lax.fori_loop(0, IDXW // 16, _q_body, 0)
                return 0

            lax.fori_loop(0, VCOL // IDXW, _r_body, 0)
            return 0

        lax.fori_loop(0, CROWS, _a_body, 0)

        for j in range(IDXROWS):
            pltpu.make_async_copy(stg.at[pl.ds(j * IDXW, IDXW)],
                                  acc_sh.at[cen_v.at[j]], ssem).start(add=True)

    _fire(0, bufs[0])
    stgs = (stg_v, stg_w)
    ssems = (ssem0, ssem1)

    def _chunk_body(it, _):
        @pl.when(lax.rem(it, 2) == 0)
        def _():
            _wait(bufs[0])

            @pl.when(it >= 1)
            def _():
                _drain(stgs[1], bufs[1][1], ssems[1])

            @pl.when(it + 1 < NCHUNK)
            def _():
                _fire(it + 1, bufs[1])

            _process(bufs[0], stgs[0], ssems[0])

        @pl.when(lax.rem(it, 2) == 1)
        def _():
            _wait(bufs[1])

            @pl.when(it >= 1)
            def _():
                _drain(stgs[0], bufs[0][1], ssems[0])

            @pl.when(it + 1 < NCHUNK)
            def _():
                _fire(it + 1, bufs[0])

            _process(bufs[1], stgs[1], ssems[1])

        return 0

    lax.fori_loop(0, NCHUNK, _chunk_body, 0)
    _drain(stgs[(NCHUNK - 1) % 2], bufs[(NCHUNK - 1) % 2][1],
           ssems[(NCHUNK - 1) % 2])

    plsc.subcore_barrier()
    pltpu.sync_copy(acc_sh.at[pl.ds(r0, ROWS_PER_TILE)],
                    out_hbm.at[cid, pl.ds(r0, ROWS_PER_TILE)])


@functools.cache
def _make_sc_scatter():
    return pl.kernel(
        _sc_body,
        out_type=jax.ShapeDtypeStruct((NC, N_PAD, 16), jnp.float32),
        mesh=plsc.VectorSubcoreMesh(core_axis_name="c", subcore_axis_name="s",
                                    num_cores=NC, num_subcores=NS),
        scratch_types=[
            pltpu.VMEM((N_PAD // 4,), jnp.int32),
            pltpu.VMEM((640,), jnp.float32),
            pltpu.VMEM((CROWS * PF, VCOL), jnp.float32),
            pltpu.VMEM((CROWS * PF, VCOL), jnp.float32),
            pltpu.VMEM((IDXROWS, IDXW), jnp.int32),
            pltpu.VMEM((IDXROWS, IDXW), jnp.int32),
            pltpu.VMEM((IDXROWS, IDXW), jnp.int32),
            pltpu.VMEM((IDXROWS, IDXW), jnp.int32),
            pltpu.VMEM((CHUNK, 16), jnp.float32),
            pltpu.VMEM((CHUNK, 16), jnp.float32),
            pltpu.VMEM_SHARED((N_PAD, 16), jnp.float32),
            pltpu.SemaphoreType.DMA,
            pltpu.SemaphoreType.DMA,
            pltpu.SemaphoreType.DMA,
            pltpu.SemaphoreType.DMA,
        ],
        compiler_params=pltpu.CompilerParams(needs_layout_passes=False,
                                             use_tc_tiling_on_sc=False),
    )


def _combine_body(p_ref, o_ref):
    o_ref[...] = p_ref[0] + p_ref[1]


def _combine(parts):
    rows = N_PAD // 8
    bn8 = rows // 16
    flat = parts.reshape(NC, rows, 128)
    out = pl.pallas_call(
        _combine_body,
        grid=(16,),
        in_specs=[pl.BlockSpec((2, bn8, 128), lambda i: (0, i, 0))],
        out_specs=pl.BlockSpec((bn8, 128), lambda i: (i, 0)),
        out_shape=jax.ShapeDtypeStruct((rows, 128), jnp.float32),
    )(flat)
    return out.reshape(N_PAD, 16)


def kernel(interatomic_vectors, centers, neighbors, species, sample_values,
           neighbor_embed, center_embed, W):
    del sample_values
    ce4 = center_embed.reshape(5, 8, 4)
    w4 = W.reshape(3, 8, 4)
    g = jnp.einsum("tj,sij,bij->tsbi", neighbor_embed, ce4, w4)
    g_flat = jnp.pad(g.reshape(-1), (0, 640 - 600)).astype(jnp.float32)

    vt3 = jnp.pad(interatomic_vectors.T,
                  ((0, 0), (0, E_PAD - N_EDGES))).reshape(3, VROWS, VCOL)
    p = _edge_feats(vt3)

    cen2 = jnp.pad(centers, (0, E_PAD - N_EDGES)).reshape(E_PAD // IDXW, IDXW)
    nei2 = jnp.pad(neighbors, (0, E_PAD - N_EDGES)).reshape(E_PAD // IDXW, IDXW)
    sp4 = jnp.pad(species, (0, N_PAD - N_ATOMS)).reshape(N_PAD // 4, 4)
    sp_packed = (sp4[:, 0] | (sp4[:, 1] << 8) | (sp4[:, 2] << 16)
                 | (sp4[:, 3] << 24))

    parts = _make_sc_scatter()(p, cen2, nei2, sp_packed, g_flat)
    return _combine(parts)[:N_ATOMS, :9].reshape(N_ATOMS, 3, 3)


# speedup vs baseline: 1.5269x; 1.0042x over previous
import functools

import jax
import jax.numpy as jnp
import numpy as np
from jax import lax
from jax.experimental import pallas as pl
from jax.experimental.pallas import tpu as pltpu
from jax.experimental.pallas import tpu_sc as plsc

N_ATOMS = 50000
N_EDGES = 800000
CUTOFF = 5.0
WIDTH = 0.5

NC = 2
NS = 16
NW = NC * NS

E_PAD = 819200
EW = E_PAD // NW
CHUNK = 1024
IDXW = 128
IDXROWS = CHUNK // IDXW
NCHUNK = EW // CHUNK

VCOL = 512
VROWS = E_PAD // VCOL
CROWS = CHUNK // VCOL

N_PAD = 50176
ROWS_PER_TILE = N_PAD // NS

BROW = 160
PF = 12


def _edge_feat_body(v_ref, p_ref):
    vx = v_ref[0]
    vy = v_ref[1]
    vz = v_ref[2]
    r2 = vx * vx + vy * vy + vz * vz + 1e-12
    inv_r2 = 1.0 / r2
    r = jnp.sqrt(r2)
    theta = (jnp.pi / CUTOFF) * jnp.minimum(r, CUTOFF)
    u = theta - (jnp.pi / 2)
    u2 = u * u
    sin_u = u * (1.0 + u2 * (-1.6666667e-1 + u2 * (8.3333331e-3
            + u2 * (-1.9840874e-4 + u2 * 2.7525562e-6))))
    cos_u = 1.0 + u2 * (-0.5 + u2 * (4.1666668e-2 + u2 * (-1.3888889e-3
            + u2 * (2.4801587e-5 + u2 * -2.7557319e-7))))
    s1 = cos_u
    c1 = -sin_u
    two_c1 = 2.0 * c1
    sines = [s1, two_c1 * s1]
    for _ in range(6):
        sines.append(two_c1 * sines[-1] - sines[-2])
    s5 = sines[4]
    fc = jnp.where(r < CUTOFF - WIDTH, 1.0, s5 * s5)
    fc = jnp.where(r < CUTOFF, fc, 0.0)
    scale = inv_r2 * fc
    rows = jnp.stack(
        [s * scale for s in sines]
        + [vx, vy, vz, jnp.zeros((BROW, VCOL), jnp.float32)], axis=1)
    p_ref[...] = rows.reshape(BROW * PF, VCOL)


def _edge_feats(vt3):
    return pl.pallas_call(
        _edge_feat_body,
        grid=(VROWS // BROW,),
        in_specs=[pl.BlockSpec((3, BROW, VCOL), lambda i: (0, i, 0))],
        out_specs=pl.BlockSpec((BROW * PF, VCOL), lambda i: (i, 0)),
        out_shape=jax.ShapeDtypeStruct((VROWS * PF, VCOL), jnp.float32),
    )(vt3)


def _sc_body(p_hbm, cen_hbm, nei_hbm, sp_hbm, g_hbm, out_hbm,
             sp_v, g_v, p_v0, p_v1, cen_v0, cen_v1, nei_v0, nei_v1,
             stg_v, stg_w, acc_sh, dsem0, dsem1, ssem0, ssem1):
    cid = lax.axis_index("c")
    sid = lax.axis_index("s")
    wid = sid * NC + cid

    pltpu.sync_copy(sp_hbm, sp_v)
    pltpu.sync_copy(g_hbm, g_v)

    zero16 = jnp.zeros((16,), jnp.float32)

    def _zero_body(i, _):
        stg_v[i, :] = zero16
        stg_w[i, :] = zero16
        return 0

    lax.fori_loop(0, CHUNK, _zero_body, 0)

    quarter = ROWS_PER_TILE // 4
    r0 = sid * ROWS_PER_TILE
    for k in range(4):
        pltpu.sync_copy(stg_v.at[pl.ds(0, quarter)],
                        acc_sh.at[pl.ds(r0 + k * quarter, quarter)])
    plsc.subcore_barrier()

    lane = lax.iota(jnp.int32, 16)
    bufs = ((p_v0, cen_v0, nei_v0, dsem0), (p_v1, cen_v1, nei_v1, dsem1))

    def _fire(it, buf):
        p_v, cen_v, nei_v, sem = buf
        base = pl.multiple_of(wid * EW + it * CHUNK, CHUNK)
        pr0 = pl.multiple_of((base // VCOL) * PF, CROWS * PF)
        row0 = pl.multiple_of(base // IDXW, IDXROWS)
        pltpu.make_async_copy(
            p_hbm.at[pl.ds(pr0, CROWS * PF)], p_v, sem).start()
        pltpu.make_async_copy(
            cen_hbm.at[pl.ds(row0, IDXROWS)], cen_v, sem).start()
        pltpu.make_async_copy(
            nei_hbm.at[pl.ds(row0, IDXROWS)], nei_v, sem).start()

    def _wait(buf):
        p_v, cen_v, nei_v, sem = buf
        pltpu.make_async_copy(
            p_hbm.at[pl.ds(0, CROWS * PF)], p_v, sem).wait()
        pltpu.make_async_copy(
            cen_hbm.at[pl.ds(0, IDXROWS)], cen_v, sem).wait()
        pltpu.make_async_copy(
            nei_hbm.at[pl.ds(0, IDXROWS)], nei_v, sem).wait()

    def _drain(stg, cen_v, ssem):
        for j in range(IDXROWS):
            pltpu.make_async_copy(stg.at[pl.ds(j * IDXW, IDXW)],
                                  acc_sh.at[cen_v.at[j]], ssem).wait()

    def _process(buf, stg, ssem):
        p_v, cen_v, nei_v, _ = buf

        def _a_body(a, _):
            def _r_body(r8, _):
                def _q_body(q, _):
                    colv = r8 * IDXW + q * 16
                    j = a * 4 + r8
                    off = q * 16
                    e0 = a * VCOL + colv
                    nvec = nei_v[j, pl.ds(off, 16)]
                    cvec = cen_v[j, pl.ds(off, 16)]
                    t_w = plsc.load_gather(
                        sp_v, [lax.shift_right_logical(nvec, 2)])
                    s_w = plsc.load_gather(
                        sp_v, [lax.shift_right_logical(cvec, 2)])
                    t_sp = lax.shift_right_logical(
                        t_w, lax.shift_left(nvec & 3, 3)) & 7
                    s_sp = lax.shift_right_logical(
                        s_w, lax.shift_left(cvec & 3, 3)) & 7
                    u24 = (t_sp * 5 + s_sp) * 24
                    col = pl.ds(colv, 16)
                    a16 = a * PF
                    rfc = [p_v[a16 + i, col] for i in range(8)]
                    ys = []
                    for b in range(3):
                        acc = rfc[0] * plsc.load_gather(g_v, [u24 + (b * 8)])
                        for i in range(1, 8):
                            acc = acc + rfc[i] * plsc.load_gather(
                                g_v, [u24 + (b * 8 + i)])
                        ys.append(acc)
                    rows = e0 + lane
                    for m in range(3):
                        d = p_v[a16 + 8 + m, col]
                        for b in range(3):
                            plsc.store_scatter(
                                stg,
                                [rows, jnp.full((16,), m * 3 + b, jnp.int32)],
                                d * ys[b])
                    return 0

                lax.fori_loop(0, IDXW // 16, _q_body, 0)
                return 0

            lax.fori_loop(0, VCOL // IDXW, _r_body, 0)
            return 0

        lax.fori_loop(0, CROWS, _a_body, 0)

        for j in range(IDXROWS):
            pltpu.make_async_copy(stg.at[pl.ds(j * IDXW, IDXW)],
                                  acc_sh.at[cen_v.at[j]], ssem).start(add=True)

    _fire(0, bufs[0])
    stgs = (stg_v, stg_w)
    ssems = (ssem0, ssem1)

    def _chunk_body(it, _):
        @pl.when(lax.rem(it, 2) == 0)
        def _():
            _wait(bufs[0])

            @pl.when(it >= 1)
            def _():
                _drain(stgs[1], bufs[1][1], ssems[1])

            @pl.when(it + 1 < NCHUNK)
            def _():
                _fire(it + 1, bufs[1])

            _process(bufs[0], stgs[0], ssems[0])

        @pl.when(lax.rem(it, 2) == 1)
        def _():
            _wait(bufs[1])

            @pl.when(it >= 1)
            def _():
                _drain(stgs[0], bufs[0][1], ssems[0])

            @pl.when(it + 1 < NCHUNK)
            def _():
                _fire(it + 1, bufs[0])

            _process(bufs[1], stgs[1], ssems[1])

        return 0

    lax.fori_loop(0, NCHUNK, _chunk_body, 0)
    _drain(stgs[(NCHUNK - 1) % 2], bufs[(NCHUNK - 1) % 2][1],
           ssems[(NCHUNK - 1) % 2])

    plsc.subcore_barrier()
    pltpu.sync_copy(acc_sh.at[pl.ds(r0, ROWS_PER_TILE)],
                    out_hbm.at[cid, pl.ds(r0, ROWS_PER_TILE)])


@functools.cache
def _make_sc_scatter():
    return pl.kernel(
        _sc_body,
        out_type=jax.ShapeDtypeStruct((NC, N_PAD, 16), jnp.float32),
        mesh=plsc.VectorSubcoreMesh(core_axis_name="c", subcore_axis_name="s",
                                    num_cores=NC, num_subcores=NS),
        scratch_types=[
            pltpu.VMEM((N_PAD // 4,), jnp.int32),
            pltpu.VMEM((640,), jnp.float32),
            pltpu.VMEM((CROWS * PF, VCOL), jnp.float32),
            pltpu.VMEM((CROWS * PF, VCOL), jnp.float32),
            pltpu.VMEM((IDXROWS, IDXW), jnp.int32),
            pltpu.VMEM((IDXROWS, IDXW), jnp.int32),
            pltpu.VMEM((IDXROWS, IDXW), jnp.int32),
            pltpu.VMEM((IDXROWS, IDXW), jnp.int32),
            pltpu.VMEM((CHUNK, 16), jnp.float32),
            pltpu.VMEM((CHUNK, 16), jnp.float32),
            pltpu.VMEM_SHARED((N_PAD, 16), jnp.float32),
            pltpu.SemaphoreType.DMA,
            pltpu.SemaphoreType.DMA,
            pltpu.SemaphoreType.DMA,
            pltpu.SemaphoreType.DMA,
        ],
        compiler_params=pltpu.CompilerParams(needs_layout_passes=False,
                                             use_tc_tiling_on_sc=False),
    )


def _combine_body(p_ref, o_ref):
    o_ref[...] = p_ref[0] + p_ref[1]


def _combine(parts):
    rows = N_PAD // 8
    bn8 = rows // 16
    flat = parts.reshape(NC, rows, 128)
    out = pl.pallas_call(
        _combine_body,
        grid=(16,),
        in_specs=[pl.BlockSpec((2, bn8, 128), lambda i: (0, i, 0))],
        out_specs=pl.BlockSpec((bn8, 128), lambda i: (i, 0)),
        out_shape=jax.ShapeDtypeStruct((rows, 128), jnp.float32),
    )(flat)
    return out.reshape(N_PAD, 16)


def kernel(interatomic_vectors, centers, neighbors, species, sample_values,
           neighbor_embed, center_embed, W):
    del sample_values
    ce4 = center_embed.reshape(5, 8, 4)
    w4 = W.reshape(3, 8, 4)
    g = jnp.einsum("tj,sij,bij->tsbi", neighbor_embed, ce4, w4)
    g_flat = jnp.pad(g.reshape(-1), (0, 640 - 600)).astype(jnp.float32)

    vt3 = jnp.pad(interatomic_vectors.T,
                  ((0, 0), (0, E_PAD - N_EDGES))).reshape(3, VROWS, VCOL)
    p = _edge_feats(vt3)

    cen2 = jnp.pad(centers, (0, E_PAD - N_EDGES)).reshape(E_PAD // IDXW, IDXW)
    nei2 = jnp.pad(neighbors, (0, E_PAD - N_EDGES)).reshape(E_PAD // IDXW, IDXW)
    sp4 = jnp.pad(species, (0, N_PAD - N_ATOMS)).reshape(N_PAD // 4, 4)
    sp_packed = (sp4[:, 0] | (sp4[:, 1] << 8) | (sp4[:, 2] << 16)
                 | (sp4[:, 3] << 24))

    parts = _make_sc_scatter()(p, cen2, nei2, sp_packed, g_flat)
    return _combine(parts)[:N_ATOMS, :9].reshape(N_ATOMS, 3, 3)
